# Initial kernel scaffold; baseline (speedup 1.0000x reference)
import jax
import jax.numpy as jnp
from jax.experimental import pallas as pl


def kernel(x, batch, node_attr, edge_src, edge_dst, emb, Wup_inv, Wup_vec, Wdown, h, mix, Wr1, br1, Wr2, br2, A1, A2, B1, B2, Ss, Sv, Cn, Cv):
    raise NotImplementedError("write your pallas kernel here")


# speedup vs baseline: 7.7248x; 7.7248x over previous
import functools
import math

import jax
import jax.numpy as jnp
from jax import lax
from jax.experimental import pallas as pl
from jax.experimental.pallas import tpu as pltpu

NS = 32
NV = 16
NB = 8
MAXR = 2.5
LAYERS = 2
NNEI = 16.0

E_BLK = 3200


def _edge_block_body(gxs_ref, gxd_ref, gv_ref, gs_ref,
                     wr1_ref, br1_ref, wr2_ref, br2_ref,
                     a1_ref, a2_ref, b1_ref, b2_ref,
                     msg_ref):
    gxs = gxs_ref[...]
    gxd = gxd_ref[...]
    gv = gv_ref[...]
    gs = gs_ref[...]

    sq3 = math.sqrt(3.0)
    n = (lax.iota(jnp.int32, NB).astype(jnp.float32) + 1.0)[None, :]
    bscale = math.sqrt(2.0 / MAXR) * math.sqrt(float(NB))

    def geom(j):
        evd = gxs[:, 3 * j:3 * j + 3] - gxd[:, 3 * j:3 * j + 3]
        elen = jnp.sqrt(jnp.sum(evd * evd, axis=1, keepdims=True) + 1e-12)
        sh = sq3 * evd / elen
        bessel = bscale * jnp.sin(n * (jnp.pi / MAXR) * elen) / elen
        u = 2.0 * (elen / MAXR - 1.0)
        c = jnp.where(u > 0, 0.0,
                      jnp.where(u < -1.0, 1.0, (1.0 - jnp.cos(jnp.pi * u)) / 2.0))
        return bessel, c * sh

    bes1, csh1 = geom(0)
    bes2, csh2 = geom(1)
    ef = jnp.concatenate([bes1, bes2], axis=1)

    pre = jnp.dot(ef, wr1_ref[...], preferred_element_type=jnp.float32) + br1_ref[...]
    silu = pre / (1.0 + jnp.exp(-pre))
    w = jnp.dot(silu, wr2_ref[...], preferred_element_type=jnp.float32) + br2_ref[...]

    d1 = (gv[:, 0:16] * csh1[:, 0:1] + gv[:, 16:32] * csh1[:, 1:2]
          + gv[:, 32:48] * csh1[:, 2:3])
    d2 = (gv[:, 0:16] * csh2[:, 0:1] + gv[:, 16:32] * csh2[:, 1:2]
          + gv[:, 32:48] * csh2[:, 2:3])
    msg_s = w[:, 0:1] * (jnp.dot(d1, a1_ref[...], preferred_element_type=jnp.float32)
                         + jnp.dot(d2, a2_ref[...], preferred_element_type=jnp.float32))

    sb1 = jnp.dot(gs, b1_ref[...], preferred_element_type=jnp.float32)
    sb2 = jnp.dot(gs, b2_ref[...], preferred_element_type=jnp.float32)
    w1 = w[:, 1:2]
    w2 = w[:, 2:3]
    mv = [w1 * sb1 * csh1[:, d:d + 1] + w2 * sb2 * csh2[:, d:d + 1] for d in range(3)]
    msg_ref[...] = jnp.concatenate([msg_s] + mv, axis=1)


def _edge_compute(gxs, gxd, gv, gs, wr1, br1, wr2, br2, a1, a2, b1, b2):
    E = gxs.shape[0]
    grid = E // E_BLK
    bspec = lambda w: pl.BlockSpec((E_BLK, w), lambda i: (i, 0))
    wspec = lambda *s: pl.BlockSpec(s, lambda i: tuple(0 for _ in s))
    return pl.pallas_call(
        _edge_block_body,
        grid=(grid,),
        in_specs=[bspec(6), bspec(6), bspec(48), bspec(32),
                  wspec(16, 32), wspec(32), wspec(32, 3), wspec(3),
                  wspec(16, 48), wspec(16, 48), wspec(32, 16), wspec(32, 16)],
        out_specs=bspec(96),
        out_shape=jax.ShapeDtypeStruct((E, 96), jnp.float32),
    )(gxs, gxd, gv, gs, wr1, br1, wr2, br2, a1, a2, b1, b2)


def kernel(x, batch, node_attr, edge_src, edge_dst, emb, Wup_inv, Wup_vec, Wdown,
           h, mix, Wr1, br1, Wr2, br2, A1, A2, B1, B2, Ss, Sv, Cn, Cv):
    N = x.shape[0]
    node_emb = jnp.take(emb, node_attr, axis=0)
    vecs = x.reshape(N, 2, 3)
    inv = jnp.stack([jnp.sum(vecs[:, 0] ** 2, -1), jnp.sum(vecs[:, 1] ** 2, -1),
                     jnp.sum(vecs[:, 0] * vecs[:, 1], -1)], axis=-1)
    s = inv @ Wup_inv
    v = jnp.einsum('nkd,kc->ncd', vecs, Wup_vec)
    v48 = jnp.transpose(v, (0, 2, 1)).reshape(N, 3 * NV)
    s_old, v_old = s, v48
    x_cur = x
    inv_sqrt_nnei = 1.0 / math.sqrt(NNEI)
    for i in range(LAYERS):
        gxs = jnp.take(x_cur, edge_src, axis=0)
        gxd = jnp.take(x_cur, edge_dst, axis=0)
        gv = jnp.take(v48, edge_src, axis=0)
        gs = jnp.take(s, edge_src, axis=0)
        msg = _edge_compute(gxs, gxd, gv, gs, Wr1[i], br1[i], Wr2[i], br2[i],
                            A1[i], A2[i], B1[i], B2[i])
        agg = jnp.zeros((N, 96), jnp.float32).at[edge_dst].add(msg) * inv_sqrt_nnei
        agg_s = agg[:, :NS + NV] + node_emb @ Cn[i]
        embcv = node_emb @ Cv[i]
        agg_v = agg[:, NS + NV:] + jnp.concatenate(
            [embcv * v48[:, 16 * d:16 * (d + 1)] for d in range(3)], axis=1)
        ynew_s = jax.nn.silu(agg_s[:, :NS])
        gates = jax.nn.sigmoid(agg_s[:, NS:])
        ynew_v = jnp.concatenate([gates * agg_v[:, 16 * d:16 * (d + 1)]
                                  for d in range(3)], axis=1)
        y2_s = s @ Ss[i]
        y2_v = jnp.concatenate([v48[:, 16 * d:16 * (d + 1)] @ Sv[i]
                                for d in range(3)], axis=1)
        tmp_s, tmp_v = s, v48
        s = 2 * s - s_old + h[i] ** 2 * (mix[i] * ynew_s + (mix[i] - 1.0) * y2_s)
        v48 = 2 * v48 - v_old + h[i] ** 2 * (mix[i] * ynew_v + (mix[i] - 1.0) * y2_v)
        s_old, v_old = tmp_s, tmp_v
        xkd = [v48[:, 16 * d:16 * (d + 1)] @ Wdown for d in range(3)]
        x_cur = jnp.stack([xkd[d][:, k] for k in range(2) for d in range(3)], axis=1)
    return (x_cur, jnp.zeros((), x.dtype))


# speedup vs baseline: 10.5865x; 1.3705x over previous
import functools
import math

import jax
import jax.numpy as jnp
from jax import lax
from jax.experimental import pallas as pl
from jax.experimental.pallas import tpu as pltpu
from jax.experimental.pallas import tpu_sc as plsc

NS = 32
NV = 16
NB = 8
MAXR = 2.5
LAYERS = 2
NNEI = 16.0

E_BLK = 3200
GW = 128


def _edge_block_body(gsrc_ref, gdstx_ref,
                     wr1_ref, br1_ref, wr2_ref, br2_ref,
                     a1_ref, a2_ref, b1_ref, b2_ref,
                     *m_refs):
    gsrc = gsrc_ref[...]
    gxs = gsrc[:, 0:6]
    gs = gsrc[:, 8:40]
    gv = gsrc[:, 40:88]
    gxd = gdstx_ref[...][:, 0:6]

    sq3 = math.sqrt(3.0)
    n = (lax.iota(jnp.int32, NB).astype(jnp.float32) + 1.0)[None, :]
    bscale = math.sqrt(2.0 / MAXR) * math.sqrt(float(NB))

    def geom(j):
        evd = gxs[:, 3 * j:3 * j + 3] - gxd[:, 3 * j:3 * j + 3]
        elen = jnp.sqrt(jnp.sum(evd * evd, axis=1, keepdims=True) + 1e-12)
        sh = sq3 * evd / elen
        bessel = bscale * jnp.sin(n * (jnp.pi / MAXR) * elen) / elen
        u = 2.0 * (elen / MAXR - 1.0)
        c = jnp.where(u > 0, 0.0,
                      jnp.where(u < -1.0, 1.0, (1.0 - jnp.cos(jnp.pi * u)) / 2.0))
        return bessel, c * sh

    bes1, csh1 = geom(0)
    bes2, csh2 = geom(1)
    ef = jnp.concatenate([bes1, bes2], axis=1)

    pre = jnp.dot(ef, wr1_ref[...], preferred_element_type=jnp.float32) + br1_ref[...]
    silu = pre / (1.0 + jnp.exp(-pre))
    w = jnp.dot(silu, wr2_ref[...], preferred_element_type=jnp.float32) + br2_ref[...]

    d1 = (gv[:, 0:16] * csh1[:, 0:1] + gv[:, 16:32] * csh1[:, 1:2]
          + gv[:, 32:48] * csh1[:, 2:3])
    d2 = (gv[:, 0:16] * csh2[:, 0:1] + gv[:, 16:32] * csh2[:, 1:2]
          + gv[:, 32:48] * csh2[:, 2:3])
    msg_s = w[:, 0:1] * (jnp.dot(d1, a1_ref[...], preferred_element_type=jnp.float32)
                         + jnp.dot(d2, a2_ref[...], preferred_element_type=jnp.float32))

    sb1 = jnp.dot(gs, b1_ref[...], preferred_element_type=jnp.float32)
    sb2 = jnp.dot(gs, b2_ref[...], preferred_element_type=jnp.float32)
    w1 = w[:, 1:2]
    w2 = w[:, 2:3]
    mv = [w1 * sb1 * csh1[:, d:d + 1] + w2 * sb2 * csh2[:, d:d + 1] for d in range(3)]
    m_refs[0][...] = jnp.concatenate([msg_s] + mv, axis=1)


def _edge_compute(gsrc, gdstx, wr1, br1, wr2, br2, a1, a2, b1, b2):
    E = gsrc.shape[0]
    grid = E // E_BLK
    bspec = lambda w: pl.BlockSpec((E_BLK, w), lambda i: (i, 0))
    wspec = lambda *s: pl.BlockSpec(s, lambda i: tuple(0 for _ in s))
    return pl.pallas_call(
        _edge_block_body,
        grid=(grid,),
        in_specs=[bspec(128), bspec(128),
                  wspec(16, 32), wspec(32), wspec(32, 3), wspec(3),
                  wspec(16, 48), wspec(16, 48), wspec(32, 16), wspec(32, 16)],
        out_specs=[bspec(96)],
        out_shape=[jax.ShapeDtypeStruct((E, 96), jnp.float32)],
    )(gsrc, gdstx, wr1, br1, wr2, br2, a1, a2, b1, b2)[0]


@functools.lru_cache(maxsize=None)
def _make_sc_gather(E, N):
    mesh = plsc.VectorSubcoreMesh(core_axis_name="c", subcore_axis_name="s")
    EPW = E // 32
    STEPS = EPW // GW

    @functools.partial(
        pl.kernel, mesh=mesh,
        out_type=(jax.ShapeDtypeStruct((E, 128), jnp.float32),
                  jax.ShapeDtypeStruct((E, 128), jnp.float32)),
        scratch_types=[
            pltpu.VMEM((GW,), jnp.int32),
            pltpu.VMEM((GW,), jnp.int32),
            pltpu.VMEM((GW, 128), jnp.float32),
            pltpu.VMEM((GW, 128), jnp.float32),
            pltpu.SemaphoreType.DMA,
            pltpu.SemaphoreType.DMA,
        ],
    )
    def gather(tbl, src_hbm, dst_hbm, gsrc_hbm, gdst_hbm,
               sidx, didx, rows, rows2, sem1, sem2):
        c = lax.axis_index("c")
        t = lax.axis_index("s")
        ebase = (t * 2 + c) * EPW

        def step(k, _):
            off = ebase + k * GW
            pltpu.sync_copy(src_hbm.at[pl.ds(off, GW)], sidx)
            pltpu.sync_copy(dst_hbm.at[pl.ds(off, GW)], didx)
            cp1 = pltpu.async_copy(tbl.at[sidx], rows, sem1)
            cp2 = pltpu.async_copy(tbl.at[didx], rows2, sem2)
            cp1.wait()
            cp2.wait()
            pltpu.sync_copy(rows, gsrc_hbm.at[pl.ds(off, GW)])
            pltpu.sync_copy(rows2, gdst_hbm.at[pl.ds(off, GW)])
            return 0

        lax.fori_loop(0, STEPS, step, 0)

    return gather


@functools.lru_cache(maxsize=None)
def _make_sc_scatter(E, N, NPT):
    mesh = plsc.VectorSubcoreMesh(core_axis_name="c", subcore_axis_name="s")
    EPT = E // 16
    STEPS = EPT // GW
    NPAD = NPT * 16

    @functools.partial(
        pl.kernel, mesh=mesh,
        out_type=jax.ShapeDtypeStruct((12, 16, NPT, 8), jnp.float32),
        scratch_types=[
            pltpu.VMEM((GW,), jnp.int32),
            pltpu.VMEM((GW, 8), jnp.float32),
            pltpu.VMEM_SHARED((NPAD, 8), jnp.float32),
        ],
    )
    def scatter(dst_hbm, m0, m1, m2, m3, m4, m5, m6, m7, m8, m9, m10, m11,
                zer_hbm, out_hbm, idx_v, msg_v, acc_sh):
        c = lax.axis_index("c")
        t = lax.axis_index("s")
        ebase = t * EPT
        planes = (m0, m1, m2, m3, m4, m5, m6, m7, m8, m9, m10, m11)
        for rnd in range(6):
            pltpu.sync_copy(zer_hbm, acc_sh.at[pl.ds(t * NPT, NPT)])
            plsc.subcore_barrier()
            for cc in range(2):
                chunk = 6 * cc + rnd

                @pl.when(c == cc)
                def _():
                    msrc = planes[chunk]

                    def step(k, _):
                        off = ebase + k * GW
                        pltpu.sync_copy(dst_hbm.at[pl.ds(off, GW)], idx_v)
                        pltpu.sync_copy(msrc.at[pl.ds(off, GW)], msg_v)
                        pltpu.sync_copy(msg_v, acc_sh.at[idx_v], add=True)
                        return 0

                    lax.fori_loop(0, STEPS, step, 0)

            plsc.subcore_barrier()
            for cc in range(2):
                chunk = 6 * cc + rnd

                @pl.when(c == cc)
                def _():
                    pltpu.sync_copy(acc_sh.at[pl.ds(t * NPT, NPT)],
                                    out_hbm.at[chunk, t])

            plsc.subcore_barrier()

    return scatter


def kernel(x, batch, node_attr, edge_src, edge_dst, emb, Wup_inv, Wup_vec, Wdown,
           h, mix, Wr1, br1, Wr2, br2, A1, A2, B1, B2, Ss, Sv, Cn, Cv):
    N = x.shape[0]
    node_emb = jnp.take(emb, node_attr, axis=0)
    vecs = x.reshape(N, 2, 3)
    inv = jnp.stack([jnp.sum(vecs[:, 0] ** 2, -1), jnp.sum(vecs[:, 1] ** 2, -1),
                     jnp.sum(vecs[:, 0] * vecs[:, 1], -1)], axis=-1)
    s = inv @ Wup_inv
    v = jnp.einsum('nkd,kc->ncd', vecs, Wup_vec)
    v48 = jnp.transpose(v, (0, 2, 1)).reshape(N, 3 * NV)
    s_old, v_old = s, v48
    x_cur = x
    inv_sqrt_nnei = 1.0 / math.sqrt(NNEI)
    E = edge_src.shape[0]
    EPAD = -(-E // 102400) * 102400
    NPT = (N // 256 + 1) * 16
    NPAD = NPT * 16
    src_p = jnp.concatenate([edge_src, jnp.zeros((EPAD - E,), jnp.int32)])
    dst_p = jnp.concatenate([edge_dst, jnp.full((EPAD - E,), N, jnp.int32)])
    zer = jnp.zeros((NPT, 8), jnp.float32)
    for i in range(LAYERS):
        pad2 = jnp.zeros((N, 2), jnp.float32)
        tbl = jnp.concatenate([x_cur, pad2, s, v48,
                               jnp.zeros((N, 40), jnp.float32)], axis=1)
        tbl = jnp.concatenate([tbl, jnp.zeros((NPAD - N, 128), jnp.float32)],
                              axis=0)
        gsrc, gdstx = _make_sc_gather(EPAD, N)(tbl, src_p, dst_p)
        msg = _edge_compute(gsrc, gdstx, Wr1[i], br1[i], Wr2[i], br2[i],
                            A1[i], A2[i], B1[i], B2[i])[:E]
        agg = jnp.zeros((N, 96), jnp.float32).at[edge_dst].add(msg) * inv_sqrt_nnei
        agg_s = agg[:, :NS + NV] + node_emb @ Cn[i]
        embcv = node_emb @ Cv[i]
        agg_v = agg[:, NS + NV:] + jnp.concatenate(
            [embcv * v48[:, 16 * d:16 * (d + 1)] for d in range(3)], axis=1)
        ynew_s = jax.nn.silu(agg_s[:, :NS])
        gates = jax.nn.sigmoid(agg_s[:, NS:])
        ynew_v = jnp.concatenate([gates * agg_v[:, 16 * d:16 * (d + 1)]
                                  for d in range(3)], axis=1)
        y2_s = s @ Ss[i]
        y2_v = jnp.concatenate([v48[:, 16 * d:16 * (d + 1)] @ Sv[i]
                                for d in range(3)], axis=1)
        tmp_s, tmp_v = s, v48
        s = 2 * s - s_old + h[i] ** 2 * (mix[i] * ynew_s + (mix[i] - 1.0) * y2_s)
        v48 = 2 * v48 - v_old + h[i] ** 2 * (mix[i] * ynew_v + (mix[i] - 1.0) * y2_v)
        s_old, v_old = tmp_s, tmp_v
        xkd = [v48[:, 16 * d:16 * (d + 1)] @ Wdown for d in range(3)]
        x_cur = jnp.stack([xkd[d][:, k] for k in range(2) for d in range(3)], axis=1)
    return (x_cur, jnp.zeros((), x.dtype))
